# baseline (speedup 1.0000x reference)
import jax
import jax.numpy as jnp
from jax import lax
from jax.experimental import pallas as pl
from jax.experimental.pallas import tpu as pltpu
from jax.experimental.pallas import tpu_sc as plsc

_B, _N = 64, 8192
_L = 16
_NW = 32
_RPW = _B // _NW
_NV = _N // _L
_U = 8


def _sc_body(x_hbm, kp1_hbm, tr_hbm, out_hbm, xv, keys, kp1v, trv, hist):
    wid = lax.axis_index("s") * 2 + lax.axis_index("c")
    base = wid * _RPW
    pltpu.sync_copy(x_hbm.at[pl.ds(base, _RPW)], xv)
    pltpu.sync_copy(kp1_hbm, kp1v)
    pltpu.sync_copy(tr_hbm, trv)
    train_nz = jnp.sum(trv[...]) != 0
    lanes = lax.iota(jnp.int32, _L)
    ones_v = jnp.ones((_L,), jnp.int32)

    def clear_hist():
        @plsc.parallel_loop(0, 256 * _U, unroll=8)
        def clr(d):
            hist[pl.ds(d * _L, _L)] = jnp.zeros((_L,), jnp.int32)

    def group_block(g0):
        block = []
        for d in range(_L):
            t = hist[pl.ds((g0 * _L + d) * _L, _L)]
            for u in range(1, _U):
                t = t + hist[pl.ds((u * 256 + g0 * _L + d) * _L, _L)]
            block.append(t)
        return block

    def scan_hist(target):
        def gsum(g, gt):
            blk = group_block(g)
            acc = blk[0]
            for d in range(1, _L):
                acc = acc + blk[d]
            return jnp.where(lanes == g, jnp.sum(acc), gt)
        gt = lax.fori_loop(0, 16, gsum, jnp.zeros((_L,), jnp.int32))
        G = plsc.cumsum(gt)
        glt = G < target
        bg = plsc.all_reduce_population_count(glt)
        gbefore = jnp.sum(jnp.where(glt & (lanes == bg - 1), G, 0))
        bg_s = jnp.sum(jnp.where(lanes == 0, bg, 0))
        t2 = target - gbefore
        blk = group_block(bg_s)
        tot = jnp.zeros((_L,), jnp.int32)
        for d in range(_L):
            tot = jnp.where(lanes == d, jnp.sum(blk[d]), tot)
        c = plsc.cumsum(tot)
        lt = c < t2
        pc = plsc.all_reduce_population_count(lt)
        cb = jnp.sum(jnp.where(lt & (lanes == pc - 1), c, 0))
        return bg * _L + pc, gbefore + cb

    for i in range(_RPW):
        r = base + i
        grp = r // _L
        lane = r % _L
        kv = kp1v[pl.ds(grp * _L, _L)]
        kp1 = jnp.sum(jnp.where(lanes == lane, kv, 0))

        clear_hist()

        zf = jnp.zeros((_L,), jnp.float32)

        @plsc.parallel_loop(0, _NV, unroll=_U, carry=(zf, zf, zf, zf))
        def p0(j, accs):
            xk = xv[i, pl.ds(j * _L, _L)]
            b = plsc.bitcast(xk, jnp.int32)
            bu = plsc.bitcast(xk, jnp.uint32)
            key = jnp.where(b < 0, ~bu, bu | jnp.uint32(0x80000000))
            keys[i, pl.ds(j * _L, _L)] = key
            digit = plsc.bitcast(key >> jnp.uint32(24), jnp.int32)
            digit = (digit + (j % _U) * 256) * _L + lanes
            plsc.addupdate_scatter(hist, [digit], ones_v)
            a0, a1, a2, a3 = accs
            a0, a1, a2, a3 = a1, a2, a3, a0 + xk
            return a0, a1, a2, a3
        num = jnp.sum(p0[0] + p0[1] + p0[2] + p0[3])

        target = jnp.zeros((_L,), jnp.int32) + kp1
        b0, cb = scan_hist(target)
        prefix = plsc.bitcast(b0, jnp.uint32) << jnp.uint32(24)
        target = target - cb

        for s in range(1, 4):
            shift = 24 - 8 * s
            himask = jnp.uint32((0xFFFFFFFF << (shift + 8)) & 0xFFFFFFFF)
            pm = prefix & himask
            clear_hist()

            @plsc.parallel_loop(0, _NV, unroll=_U)
            def dp(j):
                key = keys[i, pl.ds(j * _L, _L)]
                match = (key & himask) == pm
                digit = plsc.bitcast(
                    (key >> jnp.uint32(shift)) & jnp.uint32(0xFF), jnp.int32)
                digit = (digit + (j % _U) * 256) * _L + lanes
                plsc.addupdate_scatter(hist, [digit], ones_v, mask=match)

            bs, cb = scan_hist(target)
            prefix = prefix | (plsc.bitcast(bs, jnp.uint32)
                               << jnp.uint32(shift))
            target = target - cb

        thr_bits = jnp.where(prefix >= jnp.uint32(0x80000000),
                             prefix ^ jnp.uint32(0x80000000), ~prefix)
        thr = plsc.bitcast(plsc.bitcast(thr_bits, jnp.int32), jnp.float32)

        @plsc.parallel_loop(0, _NV, unroll=8, carry=(zf, zf, zf, zf))
        def mp(j, accs):
            xk = xv[i, pl.ds(j * _L, _L)]
            a0, a1, a2, a3 = accs
            a0, a1, a2, a3 = a1, a2, a3, a0 + jnp.where(xk < thr, 0.0, xk)
            return a0, a1, a2, a3
        num_v = jnp.zeros((_L,), jnp.float32) + num
        den_v = jnp.zeros((_L,), jnp.float32) + jnp.sum(mp[0] + mp[1] + mp[2] + mp[3])
        scale = jnp.abs(jnp.where(den_v == 0.0, 0.0, num_v / den_v))
        scale = jnp.where(train_nz, scale, 1.0)

        @plsc.parallel_loop(0, _NV, unroll=8)
        def sp(j):
            xk = xv[i, pl.ds(j * _L, _L)]
            mk = jnp.where((xk < thr) & train_nz, 0.0, xk)
            xv[i, pl.ds(j * _L, _L)] = mk * scale

    pltpu.sync_copy(xv, out_hbm.at[pl.ds(base, _RPW)])


def kernel(inputs, probs, training):
    B, N = inputs.shape
    idx = jnp.maximum(jnp.ceil(jnp.float32(N) * probs).astype(jnp.int32) - 1, 0)
    kp1 = idx + 1
    tr = jnp.full((_L,), (jnp.asarray(training) != 0).astype(jnp.int32))
    mesh = plsc.VectorSubcoreMesh(core_axis_name="c", subcore_axis_name="s")
    f = pl.kernel(
        _sc_body,
        mesh=mesh,
        compiler_params=pltpu.CompilerParams(needs_layout_passes=False),
        out_type=jax.ShapeDtypeStruct((B, N), jnp.float32),
        scratch_types=[
            pltpu.VMEM((_RPW, _N), jnp.float32),
            pltpu.VMEM((_RPW, _N), jnp.uint32),
            pltpu.VMEM((_B,), jnp.int32),
            pltpu.VMEM((_L,), jnp.int32),
            pltpu.VMEM((256 * _U * _L,), jnp.int32),
        ],
    )
    return f(inputs, kp1, tr)


# speedup vs baseline: 1.6432x; 1.6432x over previous
import jax
import jax.numpy as jnp
from jax import lax
from jax.experimental import pallas as pl
from jax.experimental.pallas import tpu as pltpu
from jax.experimental.pallas import tpu_sc as plsc

_B, _N = 64, 8192
_L = 16
_NW = 32
_RPW = _B // _NW
_NV = _N // _L
_U = 4


def _sc_body(x_hbm, kp1_hbm, tr_hbm, out_hbm, xv, keys, kp1v, trv, hist):
    wid = lax.axis_index("s") * 2 + lax.axis_index("c")
    base = wid * _RPW
    pltpu.sync_copy(x_hbm.at[pl.ds(base, _RPW)], xv)
    pltpu.sync_copy(kp1_hbm, kp1v)
    pltpu.sync_copy(tr_hbm, trv)
    train_nz = jnp.sum(trv[...]) != 0
    lanes = lax.iota(jnp.int32, _L)
    ones_v = jnp.ones((_L,), jnp.int32)

    def clear_hist(nu):
        @plsc.parallel_loop(0, 256 * nu, unroll=8)
        def clr(d):
            hist[pl.ds(d * _L, _L)] = jnp.zeros((_L,), jnp.int32)

    def group_block(g0, nu):
        block = []
        for d in range(_L):
            t = hist[pl.ds((g0 * _L + d) * _L, _L)]
            for u in range(1, nu):
                t = t + hist[pl.ds((u * 256 + g0 * _L + d) * _L, _L)]
            block.append(t)
        return block

    def scan_hist(target, nu):
        def gsum(g, gt):
            blk = group_block(g, nu)
            acc = blk[0]
            for d in range(1, _L):
                acc = acc + blk[d]
            return jnp.where(lanes == g, jnp.sum(acc), gt)
        gt = lax.fori_loop(0, 16, gsum, jnp.zeros((_L,), jnp.int32))
        G = plsc.cumsum(gt)
        glt = G < target
        bg = plsc.all_reduce_population_count(glt)
        gbefore = jnp.sum(jnp.where(glt & (lanes == bg - 1), G, 0))
        bg_s = jnp.sum(jnp.where(lanes == 0, bg, 0))
        t2 = target - gbefore
        blk = group_block(bg_s, nu)
        tot = jnp.zeros((_L,), jnp.int32)
        for d in range(_L):
            tot = jnp.where(lanes == d, jnp.sum(blk[d]), tot)
        c = plsc.cumsum(tot)
        lt = c < t2
        pc = plsc.all_reduce_population_count(lt)
        cb = jnp.sum(jnp.where(lt & (lanes == pc - 1), c, 0))
        return bg * _L + pc, gbefore + cb

    for i in range(_RPW):
        r = base + i
        grp = r // _L
        lane = r % _L
        kv = kp1v[pl.ds(grp * _L, _L)]
        kp1 = jnp.sum(jnp.where(lanes == lane, kv, 0))

        clear_hist(_U)

        zf = jnp.zeros((_L,), jnp.float32)

        @plsc.parallel_loop(0, _NV, unroll=_U, carry=(zf, zf, zf, zf))
        def p0(j, accs):
            xk = xv[i, pl.ds(j * _L, _L)]
            b = plsc.bitcast(xk, jnp.int32)
            bu = plsc.bitcast(xk, jnp.uint32)
            key = jnp.where(b < 0, ~bu, bu | jnp.uint32(0x80000000))
            keys[i, pl.ds(j * _L, _L)] = key
            digit = plsc.bitcast(key >> jnp.uint32(24), jnp.int32)
            digit = (digit + (j % _U) * 256) * _L + lanes
            plsc.addupdate_scatter(hist, [digit], ones_v)
            a0, a1, a2, a3 = accs
            a0, a1, a2, a3 = a1, a2, a3, a0 + xk
            return a0, a1, a2, a3
        num = jnp.sum(p0[0] + p0[1] + p0[2] + p0[3])

        target = jnp.zeros((_L,), jnp.int32) + kp1
        b0, cb = scan_hist(target, _U)
        prefix = plsc.bitcast(b0, jnp.uint32) << jnp.uint32(24)
        target = target - cb

        for s in range(1, 4):
            shift = 24 - 8 * s
            himask = jnp.uint32((0xFFFFFFFF << (shift + 8)) & 0xFFFFFFFF)
            pm = prefix & himask
            clear_hist(1)

            @plsc.parallel_loop(0, _NV, unroll=4)
            def dp(j):
                key = keys[i, pl.ds(j * _L, _L)]
                match = (key & himask) == pm
                digit = plsc.bitcast(
                    (key >> jnp.uint32(shift)) & jnp.uint32(0xFF), jnp.int32)
                digit = digit * _L + lanes
                plsc.addupdate_scatter(hist, [digit], ones_v, mask=match)

            bs, cb = scan_hist(target, 1)
            prefix = prefix | (plsc.bitcast(bs, jnp.uint32)
                               << jnp.uint32(shift))
            target = target - cb

        thr_bits = jnp.where(prefix >= jnp.uint32(0x80000000),
                             prefix ^ jnp.uint32(0x80000000), ~prefix)
        thr = plsc.bitcast(plsc.bitcast(thr_bits, jnp.int32), jnp.float32)

        @plsc.parallel_loop(0, _NV, unroll=8, carry=(zf, zf, zf, zf))
        def mp(j, accs):
            xk = xv[i, pl.ds(j * _L, _L)]
            a0, a1, a2, a3 = accs
            a0, a1, a2, a3 = a1, a2, a3, a0 + jnp.where(xk < thr, 0.0, xk)
            return a0, a1, a2, a3
        num_v = jnp.zeros((_L,), jnp.float32) + num
        den_v = jnp.zeros((_L,), jnp.float32) + jnp.sum(mp[0] + mp[1] + mp[2] + mp[3])
        scale = jnp.abs(jnp.where(den_v == 0.0, 0.0, num_v / den_v))
        scale = jnp.where(train_nz, scale, 1.0)

        @plsc.parallel_loop(0, _NV, unroll=8)
        def sp(j):
            xk = xv[i, pl.ds(j * _L, _L)]
            mk = jnp.where((xk < thr) & train_nz, 0.0, xk)
            xv[i, pl.ds(j * _L, _L)] = mk * scale

    pltpu.sync_copy(xv, out_hbm.at[pl.ds(base, _RPW)])


def kernel(inputs, probs, training):
    B, N = inputs.shape
    idx = jnp.maximum(jnp.ceil(jnp.float32(N) * probs).astype(jnp.int32) - 1, 0)
    kp1 = idx + 1
    tr = jnp.full((_L,), (jnp.asarray(training) != 0).astype(jnp.int32))
    mesh = plsc.VectorSubcoreMesh(core_axis_name="c", subcore_axis_name="s")
    f = pl.kernel(
        _sc_body,
        mesh=mesh,
        compiler_params=pltpu.CompilerParams(needs_layout_passes=False),
        out_type=jax.ShapeDtypeStruct((B, N), jnp.float32),
        scratch_types=[
            pltpu.VMEM((_RPW, _N), jnp.float32),
            pltpu.VMEM((_RPW, _N), jnp.uint32),
            pltpu.VMEM((_B,), jnp.int32),
            pltpu.VMEM((_L,), jnp.int32),
            pltpu.VMEM((256 * _U * _L,), jnp.int32),
        ],
    )
    return f(inputs, kp1, tr)
